# Initial kernel scaffold; baseline (speedup 1.0000x reference)
import jax
import jax.numpy as jnp
from jax.experimental import pallas as pl


def kernel(x, edge_index, Wh0_r, Wh0_n, bh0, Wg0_r, Wg0_n, bg0, Wh1_r, Wh1_n, bh1, Wg1_r, Wg1_n, bg1):
    raise NotImplementedError("write your pallas kernel here")


# speedup vs baseline: 15.8838x; 15.8838x over previous
import functools

import jax
import jax.numpy as jnp
from jax import lax
from jax.experimental import pallas as pl
from jax.experimental.pallas import tpu as pltpu
from jax.experimental.pallas import tpu_sc as plsc

N = 10000
E = 320000
D = 128
NC, NS = 2, 16
NW = NC * NS
CH = 80


def _make_agg(n_src, dst_sizes):
    per_tile = E // NW
    k = per_tile // CH
    n_dst = len(dst_sizes)

    scratch = (
        [pltpu.VMEM((k, CH), jnp.int32)]
        + [pltpu.VMEM((k, CH), jnp.int32) for _ in dst_sizes]
        + [pltpu.VMEM((CH, D), jnp.float32)]
        + [pltpu.VMEM_SHARED((nd, D), jnp.float32) for nd in dst_sizes]
        + [pltpu.SemaphoreType.DMA]
    )
    out_type = [jax.ShapeDtypeStruct((NC, nd, D), jnp.float32)
                for nd in dst_sizes]
    mesh = plsc.VectorSubcoreMesh(core_axis_name="c", subcore_axis_name="s",
                                  num_cores=NC, num_subcores=NS)

    def body(x_hbm, src_hbm, *rest):
        dst_hbm = rest[:n_dst]
        zeros_hbm = rest[n_dst]
        outs = rest[n_dst + 1:2 * n_dst + 1]
        src_v = rest[2 * n_dst + 1]
        dst_v = rest[2 * n_dst + 2:3 * n_dst + 2]
        rows_v = rest[3 * n_dst + 2]
        accs = rest[3 * n_dst + 3:4 * n_dst + 3]
        sem = rest[4 * n_dst + 3]

        c = lax.axis_index("c")
        s = lax.axis_index("s")
        w = c * NS + s

        def striped(nd, fn):
            q = 8 * (nd // (NS * 8))
            last = nd - (NS - 1) * q

            @pl.when(s < NS - 1)
            def _():
                fn(pl.ds(s * q, q))

            @pl.when(s == NS - 1)
            def _():
                fn(pl.ds((NS - 1) * q, last))

        pltpu.sync_copy(src_hbm.at[w], src_v)
        for dh, dv in zip(dst_hbm, dst_v):
            pltpu.sync_copy(dh.at[w], dv)

        for acc, nd in zip(accs, dst_sizes):
            striped(nd, lambda sl, acc=acc:
                    pltpu.sync_copy(zeros_hbm.at[sl], acc.at[sl]))
        plsc.subcore_barrier()

        def step(j, carry):
            pltpu.async_copy(x_hbm.at[src_v.at[j]], rows_v, sem).wait()
            for acc, dv in zip(accs, dst_v):
                pltpu.sync_copy(rows_v, acc.at[dv.at[j]], add=True)
            return carry
        lax.fori_loop(0, k, step, 0)
        plsc.subcore_barrier()

        for acc, out, nd in zip(accs, outs, dst_sizes):
            striped(nd, lambda sl, acc=acc, out=out:
                    pltpu.sync_copy(acc.at[sl], out.at[c, sl]))

    return pl.kernel(body, out_type=out_type, mesh=mesh,
                     scratch_types=scratch)


def _relu(v):
    return jnp.maximum(v, 0.0)


def _mm(a, b):
    return jnp.dot(a, b, preferred_element_type=jnp.float32)


def _conv_pool_body(x_ref, a0_ref, a1_ref, wr_ref, wn_ref, b_ref, p_ref):
    h = _relu(_mm(x_ref[...], wr_ref[...])
              + _mm(a0_ref[...] + a1_ref[...], wn_ref[...]) + b_ref[...])
    m = p_ref.shape[0]
    p_ref[...] = h.reshape(m, 2, D).sum(axis=1)


def _lift_conv_body(t_ref, a0_ref, a1_ref, wr_ref, wn_ref, b_ref, o_ref):
    m = t_ref.shape[0]
    t = _mm(t_ref[...], wr_ref[...]) * 0.5
    y = jnp.broadcast_to(t[:, None, :], (m, 2, D)).reshape(2 * m, D)
    agg = (a0_ref[...] + a1_ref[...]) * 0.5
    o_ref[...] = _relu(y + _mm(agg, wn_ref[...]) + b_ref[...])


def _mid_body(p0_ref, ay0_ref, ay1_ref, whr_ref, whn_ref, bh_ref,
              wgr_ref, wgn_ref, bg_ref, p1_ref, o1_ref):
    ay = ay0_ref[...] + ay1_ref[...]
    agg1 = ay.reshape(N // 2, 2, D).sum(axis=1)
    p0 = p0_ref[...]
    h1 = _relu(_mm(p0, whr_ref[...]) + _mm(agg1, whn_ref[...]) + bh_ref[...])
    p1_ref[...] = h1.reshape(N // 4, 2, D).sum(axis=1)
    t = _mm(p0, wgr_ref[...]) * 0.5
    y = jnp.broadcast_to(t[:, None, :], (N // 2, 2, D)).reshape(N, D)
    o1_ref[...] = _relu(y + _mm(ay * 0.5, wgn_ref[...]) + bg_ref[...])


def _conv_pool(x, a0, a1, wr, wn, b, n_out):
    return pl.pallas_call(
        _conv_pool_body,
        out_shape=jax.ShapeDtypeStruct((n_out, D), jnp.float32),
    )(x, a0, a1, wr, wn, b.reshape(1, D))


def _lift_conv(t, a0, a1, wr, wn, b):
    return pl.pallas_call(
        _lift_conv_body,
        out_shape=jax.ShapeDtypeStruct((2 * t.shape[0], D), jnp.float32),
    )(t, a0, a1, wr, wn, b.reshape(1, D))


@functools.partial(jax.jit, static_argnames=())
def kernel(x, edge_index, Wh0_r, Wh0_n, bh0, Wg0_r, Wg0_n, bg0,
           Wh1_r, Wh1_n, bh1, Wg1_r, Wg1_n, bg1):
    src = edge_index[0].astype(jnp.int32)
    dst = edge_index[1].astype(jnp.int32)
    src1 = src // 2
    dst1 = dst // 2
    src2 = src // 4
    kshape = (NW, E // (NW * CH), CH)
    src_r = src.reshape(kshape)
    dst_r = dst.reshape(kshape)
    src1_r = src1.reshape(kshape)
    dst1_r = dst1.reshape(kshape)
    src2_r = src2.reshape(kshape)
    zeros = jnp.zeros((N, D), jnp.float32)

    (a0,) = _make_agg(N, (N,))(x, src_r, dst_r, zeros)
    p0 = _conv_pool(x, a0[0], a0[1], Wh0_r, Wh0_n, bh0, N // 2)

    (ay,) = _make_agg(N // 2, (N,))(p0, src1_r, dst_r, zeros)
    p1, out1 = pl.pallas_call(
        _mid_body,
        out_shape=(jax.ShapeDtypeStruct((N // 4, D), jnp.float32),
                   jax.ShapeDtypeStruct((N, D), jnp.float32)),
    )(p0, ay[0], ay[1], Wh1_r, Wh1_n, bh1.reshape(1, D),
      Wg0_r, Wg0_n, bg0.reshape(1, D))

    (az,) = _make_agg(N // 4, (N // 2,))(p1, src2_r, dst1_r, zeros)
    z2 = _lift_conv(p1, az[0], az[1], Wg1_r, Wg1_n, bg1)
    (aw,) = _make_agg(N // 2, (N,))(z2, src1_r, dst_r, zeros)
    out2 = _lift_conv(z2, aw[0], aw[1], Wg0_r, Wg0_n, bg0)

    return (jnp.stack([x, out1, out2], axis=1), x, p0, p1)
